# baseline (speedup 1.0000x reference)
import functools

import jax
import jax.numpy as jnp
from jax import lax
from jax.experimental import pallas as pl
from jax.experimental.pallas import tpu as pltpu
from jax.experimental.pallas import tpu_sc as plsc

N_NODES = 10000
N_EDGES = 320000
D = 128

NC = 2
NS = 16
L = 16
NW = NC * NS
NPW = N_EDGES // NW
C = 80
NCHUNK = NPW // C
NGRP = C // L


def _body(x_hbm, src_hbm, dst_hbm, neg_hbm, outp_hbm, outn_hbm,
          idx_s, idx_d, idx_n,
          sA, dA, nA, sB, dB, nB,
          op_v, on_v, scr_p, scr_n, semA, semB):
  wid = lax.axis_index("s") * NC + lax.axis_index("c")
  base = pl.multiple_of(wid * NPW, 8)

  pltpu.sync_copy(src_hbm.at[pl.ds(base, NPW)], idx_s)
  pltpu.sync_copy(dst_hbm.at[pl.ds(base, NPW)], idx_d)
  pltpu.sync_copy(neg_hbm.at[pl.ds(base, NPW)], idx_n)

  lane = lax.iota(jnp.int32, L)

  def start_gather(g, s_buf, d_buf, n_buf, sem):
    off = pl.multiple_of(g * C, 8)
    pltpu.make_async_copy(
        x_hbm.at[idx_s.at[pl.ds(off, C)]], s_buf, sem).start()
    pltpu.make_async_copy(
        x_hbm.at[idx_d.at[pl.ds(off, C)]], d_buf, sem).start()
    pltpu.make_async_copy(
        x_hbm.at[idx_n.at[pl.ds(off, C)]], n_buf, sem).start()

  def wait_gather(s_buf, d_buf, n_buf, sem):
    pltpu.make_async_copy(
        x_hbm.at[idx_s.at[pl.ds(0, C)]], s_buf, sem).wait()
    pltpu.make_async_copy(
        x_hbm.at[idx_d.at[pl.ds(0, C)]], d_buf, sem).wait()
    pltpu.make_async_copy(
        x_hbm.at[idx_n.at[pl.ds(0, C)]], n_buf, sem).wait()

  def tree_sum(vs):
    while len(vs) > 1:
      nxt = [vs[i] + vs[i + 1] for i in range(0, len(vs) - 1, 2)]
      if len(vs) % 2:
        nxt.append(vs[-1])
      vs = nxt
    return vs[0]

  def compute(g, s_buf, d_buf, n_buf):
    coff = g * C

    def grp_body(k, carry):
      base_e = k * L
      for ee in range(L):
        e = base_e + ee
        pp, nn = [], []
        for j in range(D // L):
          o = j * L
          sv = s_buf[e, pl.ds(o, L)]
          pp.append(sv * d_buf[e, pl.ds(o, L)])
          nn.append(sv * n_buf[e, pl.ds(o, L)])
        scr_p[ee, pl.ds(0, L)] = tree_sum(pp)
        scr_n[ee, pl.ds(0, L)] = tree_sum(nn)
      cols_p, cols_n = [], []
      for c in range(L):
        colc = jnp.full((L,), c, jnp.int32)
        cols_p.append(plsc.load_gather(scr_p, [lane, colc]))
        cols_n.append(plsc.load_gather(scr_n, [lane, colc]))
      o = pl.multiple_of(coff + base_e, 8)
      op_v[pl.ds(o, L)] = tree_sum(cols_p)
      on_v[pl.ds(o, L)] = tree_sum(cols_n)
      return carry

    lax.fori_loop(0, NGRP, grp_body, 0)

  start_gather(0, sA, dA, nA, semA)

  def pipe(it, carry):
    ga = 2 * it
    start_gather(ga + 1, sB, dB, nB, semB)
    wait_gather(sA, dA, nA, semA)
    compute(ga, sA, dA, nA)
    start_gather(ga + 2, sA, dA, nA, semA)
    wait_gather(sB, dB, nB, semB)
    compute(ga + 1, sB, dB, nB)
    return carry

  lax.fori_loop(0, (NCHUNK - 1) // 2, pipe, 0)
  wait_gather(sA, dA, nA, semA)
  compute(NCHUNK - 1, sA, dA, nA)

  pltpu.sync_copy(op_v, outp_hbm.at[pl.ds(base, NPW)])
  pltpu.sync_copy(on_v, outn_hbm.at[pl.ds(base, NPW)])


@jax.jit
def _dot_decoder(x, src, dst, neg):
  mesh = plsc.VectorSubcoreMesh(core_axis_name="c", subcore_axis_name="s")
  run = pl.kernel(
      _body,
      out_type=(
          jax.ShapeDtypeStruct((N_EDGES,), jnp.float32),
          jax.ShapeDtypeStruct((N_EDGES,), jnp.float32),
      ),
      mesh=mesh,
      scratch_types=[
          pltpu.VMEM((NPW,), jnp.int32),
          pltpu.VMEM((NPW,), jnp.int32),
          pltpu.VMEM((NPW,), jnp.int32),
          pltpu.VMEM((C, D), jnp.float32),
          pltpu.VMEM((C, D), jnp.float32),
          pltpu.VMEM((C, D), jnp.float32),
          pltpu.VMEM((C, D), jnp.float32),
          pltpu.VMEM((C, D), jnp.float32),
          pltpu.VMEM((C, D), jnp.float32),
          pltpu.VMEM((NPW,), jnp.float32),
          pltpu.VMEM((NPW,), jnp.float32),
          pltpu.VMEM((L, L + 1), jnp.float32),
          pltpu.VMEM((L, L + 1), jnp.float32),
          pltpu.SemaphoreType.DMA,
          pltpu.SemaphoreType.DMA,
      ],
      compiler_params=pltpu.CompilerParams(needs_layout_passes=False),
      name="dot_decoder_sc",
  )
  return run(x, src, dst, neg)


def kernel(x, edge_index, neg_dst):
  src = edge_index[0].astype(jnp.int32)
  dst = edge_index[1].astype(jnp.int32)
  neg = neg_dst.astype(jnp.int32)
  return _dot_decoder(x, src, dst, neg)


# speedup vs baseline: 1.4904x; 1.4904x over previous
import jax
import jax.numpy as jnp
from jax import lax
from jax.experimental import pallas as pl
from jax.experimental.pallas import tpu as pltpu
from jax.experimental.pallas import tpu_sc as plsc

N_NODES = 10000
N_EDGES = 320000
D = 128

NC = 2
NS = 16
L = 16
NW = NC * NS
NPW = N_EDGES // NW
C = 80
NCHUNK = NPW // C
NGRP = C // L


def _body(x_hbm, src_hbm, dst_hbm, neg_hbm, outp_hbm, outn_hbm,
          idx_s, idx_d, idx_n,
          sA, dA, nA, sB, dB, nB,
          opA, onA, opB, onB, scr_p, scr_n,
          semA, semB, semOA, semOB):
  wid = lax.axis_index("s") * NC + lax.axis_index("c")
  base = pl.multiple_of(wid * NPW, 8)

  pltpu.sync_copy(src_hbm.at[pl.ds(base, NPW)], idx_s)
  pltpu.sync_copy(dst_hbm.at[pl.ds(base, NPW)], idx_d)
  pltpu.sync_copy(neg_hbm.at[pl.ds(base, NPW)], idx_n)

  lane = lax.iota(jnp.int32, L)

  def start_gather(g, s_buf, d_buf, n_buf, sem):
    off = pl.multiple_of(g * C, 8)
    pltpu.make_async_copy(
        x_hbm.at[idx_s.at[pl.ds(off, C)]], s_buf, sem).start()
    pltpu.make_async_copy(
        x_hbm.at[idx_d.at[pl.ds(off, C)]], d_buf, sem).start()
    pltpu.make_async_copy(
        x_hbm.at[idx_n.at[pl.ds(off, C)]], n_buf, sem).start()

  def wait_gather(s_buf, d_buf, n_buf, sem):
    pltpu.make_async_copy(
        x_hbm.at[idx_s.at[pl.ds(0, C)]], s_buf, sem).wait()
    pltpu.make_async_copy(
        x_hbm.at[idx_d.at[pl.ds(0, C)]], d_buf, sem).wait()
    pltpu.make_async_copy(
        x_hbm.at[idx_n.at[pl.ds(0, C)]], n_buf, sem).wait()

  def start_store(g, op_b, on_b, sem):
    o = base + g * C
    pltpu.make_async_copy(op_b, outp_hbm.at[pl.ds(o, C)], sem).start()
    pltpu.make_async_copy(on_b, outn_hbm.at[pl.ds(o, C)], sem).start()

  def wait_store(op_b, on_b, sem):
    pltpu.make_async_copy(op_b, outp_hbm.at[pl.ds(0, C)], sem).wait()
    pltpu.make_async_copy(on_b, outn_hbm.at[pl.ds(0, C)], sem).wait()

  colbase = lane * (L + 1)

  def compute(s_buf, d_buf, n_buf, op_b, on_b):
    def grp_body(k, carry):
      base_e = k * L
      for ee in range(L):
        e = base_e + ee
        sv0 = s_buf[e, pl.ds(0, L)]
        accp = sv0 * d_buf[e, pl.ds(0, L)]
        accn = sv0 * n_buf[e, pl.ds(0, L)]
        for j in range(1, D // L):
          o = j * L
          sv = s_buf[e, pl.ds(o, L)]
          accp = accp + sv * d_buf[e, pl.ds(o, L)]
          accn = accn + sv * n_buf[e, pl.ds(o, L)]
        scr_p[pl.ds(ee * (L + 1), L)] = accp
        scr_n[pl.ds(ee * (L + 1), L)] = accn
      rp = plsc.load_gather(scr_p, [colbase])
      rn = plsc.load_gather(scr_n, [colbase])
      for c in range(1, L):
        rp = rp + plsc.load_gather(scr_p, [colbase + c])
        rn = rn + plsc.load_gather(scr_n, [colbase + c])
      op_b[pl.ds(base_e, L)] = rp
      on_b[pl.ds(base_e, L)] = rn
      return carry

    lax.fori_loop(0, NGRP, grp_body, 0)

  start_gather(0, sA, dA, nA, semA)

  def pipe(it, carry):
    ga = 2 * it
    start_gather(ga + 1, sB, dB, nB, semB)
    wait_gather(sA, dA, nA, semA)

    @pl.when(it > 0)
    def _():
      wait_store(opA, onA, semOA)

    compute(sA, dA, nA, opA, onA)
    start_store(ga, opA, onA, semOA)
    start_gather(ga + 2, sA, dA, nA, semA)
    wait_gather(sB, dB, nB, semB)

    @pl.when(it > 0)
    def _():
      wait_store(opB, onB, semOB)

    compute(sB, dB, nB, opB, onB)
    start_store(ga + 1, opB, onB, semOB)
    return carry

  lax.fori_loop(0, (NCHUNK - 1) // 2, pipe, 0)
  wait_gather(sA, dA, nA, semA)
  wait_store(opA, onA, semOA)
  compute(sA, dA, nA, opA, onA)
  start_store(NCHUNK - 1, opA, onA, semOA)
  wait_store(opA, onA, semOA)
  wait_store(opB, onB, semOB)


@jax.jit
def _dot_decoder(x, src, dst, neg):
  mesh = plsc.VectorSubcoreMesh(core_axis_name="c", subcore_axis_name="s")
  run = pl.kernel(
      _body,
      out_type=(
          jax.ShapeDtypeStruct((N_EDGES,), jnp.float32),
          jax.ShapeDtypeStruct((N_EDGES,), jnp.float32),
      ),
      mesh=mesh,
      scratch_types=[
          pltpu.VMEM((NPW,), jnp.int32),
          pltpu.VMEM((NPW,), jnp.int32),
          pltpu.VMEM((NPW,), jnp.int32),
          pltpu.VMEM((C, D), jnp.float32),
          pltpu.VMEM((C, D), jnp.float32),
          pltpu.VMEM((C, D), jnp.float32),
          pltpu.VMEM((C, D), jnp.float32),
          pltpu.VMEM((C, D), jnp.float32),
          pltpu.VMEM((C, D), jnp.float32),
          pltpu.VMEM((C,), jnp.float32),
          pltpu.VMEM((C,), jnp.float32),
          pltpu.VMEM((C,), jnp.float32),
          pltpu.VMEM((C,), jnp.float32),
          pltpu.VMEM((L * (L + 1),), jnp.float32),
          pltpu.VMEM((L * (L + 1),), jnp.float32),
          pltpu.SemaphoreType.DMA,
          pltpu.SemaphoreType.DMA,
          pltpu.SemaphoreType.DMA,
          pltpu.SemaphoreType.DMA,
      ],
      compiler_params=pltpu.CompilerParams(needs_layout_passes=False),
      name="dot_decoder_sc",
  )
  return run(x, src, dst, neg)


def kernel(x, edge_index, neg_dst):
  src = edge_index[0].astype(jnp.int32)
  dst = edge_index[1].astype(jnp.int32)
  neg = neg_dst.astype(jnp.int32)
  return _dot_decoder(x, src, dst, neg)
